# baseline (speedup 1.0000x reference)
import functools

import jax
import jax.numpy as jnp
import numpy as np
from jax import lax
from jax.experimental import pallas as pl
from jax.experimental.pallas import tpu as pltpu
from jax.experimental.pallas import tpu_sc as plsc

_BN_EPS = 1e-05
_INTERP = False
_NW = 32


def _fps_body(x_ref, y_ref, z_ref, idx_ref):
    B, S, L = x_ref.shape
    N = S * L
    M = idx_ref.shape[1]
    sub_i = lax.broadcasted_iota(jnp.int32, (B, S, L), 1)
    lane_i = lax.broadcasted_iota(jnp.int32, (B, S, L), 2)
    flat_i = sub_i * L + lane_i
    lane_m = lax.broadcasted_iota(jnp.int32, (B, M), 1)
    x = x_ref[...]
    y = y_ref[...]
    z = z_ref[...]
    idx_ref[...] = jnp.zeros((B, M), jnp.int32)

    def red(op, a):
        return op(op(a, axis=2, keepdims=True), axis=1, keepdims=True)

    def body(i, carry):
        last, dists = carry
        sel = flat_i == last
        lx = red(jnp.sum, jnp.where(sel, x, 0.0))
        ly = red(jnp.sum, jnp.where(sel, y, 0.0))
        lz = red(jnp.sum, jnp.where(sel, z, 0.0))
        dx = x - lx
        dy = y - ly
        dz = z - lz
        d = (dx * dx + dy * dy) + dz * dz
        dmin = jnp.minimum(dists, d)
        mx = red(jnp.max, dmin)
        nxt = red(jnp.min, jnp.where(dmin == mx, flat_i, N))
        idx_ref[...] += jnp.where(lane_m == i, nxt[:, 0, :], 0)
        return nxt, dmin

    lax.fori_loop(1, M, body,
                  (jnp.zeros((B, 1, 1), jnp.int32),
                   jnp.full((B, S, L), 1e10, jnp.float32)))


def _fps(x, y, z, M):
    B, N = x.shape
    x3 = x.reshape(B, N // 128, 128)
    y3 = y.reshape(B, N // 128, 128)
    z3 = z.reshape(B, N // 128, 128)
    return pl.pallas_call(
        _fps_body,
        interpret=_INTERP,
        out_shape=jax.ShapeDtypeStruct((B, M), jnp.int32),
    )(x3, y3, z3)


def _iota16():
    return lax.broadcasted_iota(jnp.int32, (16,), 0)


def _splat(v):
    return jnp.full((16,), v, jnp.int32)


def _select_ref(m, rxv, ryv, rzv, xsv, ysv, zsv, nbv, r, k, NCH):
    i16 = _iota16()
    rxs = plsc.load_gather(rxv, [_splat(m)])
    rys = plsc.load_gather(ryv, [_splat(m)])
    rzs = plsc.load_gather(rzv, [_splat(m)])
    nbv[pl.ds(0, 16)] = jnp.zeros((16,), jnp.int32)
    r2 = jnp.float32(r * r)
    U = 8
    NG = NCH // U

    def cond(st):
        cc, cnt = st
        return jnp.logical_and(cnt < k, cc < NG)

    def body(st):
        cc, cnt = st
        base = cc * (16 * U)
        cntv = jnp.full((16,), cnt, jnp.int32)
        total = jnp.zeros((16,), jnp.int32)
        msks = []
        cums = []
        pcs = []
        for u in range(U):
            off = base + u * 16
            xv = xsv[pl.ds(off, 16)]
            yv = ysv[pl.ds(off, 16)]
            zv = zsv[pl.ds(off, 16)]
            dx = rxs - xv
            dy = rys - yv
            dz = rzs - zv
            d2 = (dx * dx + dy * dy) + dz * dz
            msk = jnp.logical_not(d2 > r2)
            mi32 = msk.astype(jnp.int32)
            msks.append(msk)
            cums.append(plsc.cumsum(mi32))
            pcs.append(plsc.all_reduce_population_count(msk))
            total = total + mi32
        run = cntv
        for u in range(U):
            pos = (cums[u] - 1) + run
            okm = jnp.logical_and(msks[u], pos < k)
            plsc.store_scatter(nbv, [pos], base + u * 16 + i16, mask=okm)
            run = run + pcs[u]
        return cc + 1, cnt + jnp.sum(total)

    _, cnt = lax.while_loop(cond, body, (jnp.int32(0), jnp.int32(0)))
    return jnp.minimum(cnt, k), (rxs, rys, rzs)


def _emit_slots(mi, cnt, splats, xsv, ysv, zsv, nbv, relv, idxbv, boff, r, k):
    i16 = _iota16()
    rxs, rys, rzs = splats
    nb0 = plsc.load_gather(nbv, [jnp.zeros((16,), jnp.int32)])
    rinv = jnp.float32(r)
    z16f = jnp.zeros((16,), jnp.float32)
    for j in range(k // 16):
        slot = j * 16 + i16
        raw = nbv[pl.ds(j * 16, 16)]
        idx16 = jnp.where(slot < cnt, raw, nb0)
        xg = plsc.load_gather(xsv, [idx16])
        yg = plsc.load_gather(ysv, [idx16])
        zg = plsc.load_gather(zsv, [idx16])
        xr = (xg - rxs) / rinv
        yr = (yg - rys) / rinv
        zr = (zg - rzs) / rinv
        rows8 = (mi * k + slot) * 8
        plsc.store_scatter(relv, [rows8 + 0], xr)
        plsc.store_scatter(relv, [rows8 + 1], yr)
        plsc.store_scatter(relv, [rows8 + 2], zr)
        for ccol in range(3, 8):
            plsc.store_scatter(relv, [rows8 + ccol], z16f)
        if idxbv is not None:
            idxbv[pl.ds(mi * k + j * 16, 16)] = idx16 + boff


def _sc_scale1_body(x_h, y_h, z_h, idx_h, xrel_h, sx_h, sy_h, sz_h,
                    xsv, ysv, zsv, idxv, rxv, ryv, rzv, nbv, relv,
                    *, B, N, M, k, r, MC, RC):
    cid = lax.axis_index("c")
    sid = lax.axis_index("s")
    wid = sid * 2 + cid
    cpb = _NW // B
    b = wid // cpb
    ch = wid % cpb
    base_m = ch * MC
    pltpu.sync_copy(x_h.at[b], xsv)
    pltpu.sync_copy(y_h.at[b], ysv)
    pltpu.sync_copy(z_h.at[b], zsv)
    pltpu.sync_copy(idx_h.at[b, pl.ds(base_m, MC)], idxv)

    def gref(j, _):
        iv = idxv[pl.ds(j * 16, 16)]
        rxv[pl.ds(j * 16, 16)] = plsc.load_gather(xsv, [iv])
        ryv[pl.ds(j * 16, 16)] = plsc.load_gather(ysv, [iv])
        rzv[pl.ds(j * 16, 16)] = plsc.load_gather(zsv, [iv])
        return 0

    lax.fori_loop(0, MC // 16, gref, 0)
    pltpu.sync_copy(rxv, sx_h.at[b, pl.ds(base_m, MC)])
    pltpu.sync_copy(ryv, sy_h.at[b, pl.ds(base_m, MC)])
    pltpu.sync_copy(rzv, sz_h.at[b, pl.ds(base_m, MC)])

    NCH = N // 16
    for sub in range(MC // RC):
        def per_ref(mi, _):
            m = sub * RC + mi
            cnt, splats = _select_ref(m, rxv, ryv, rzv, xsv, ysv, zsv, nbv,
                                      r, k, NCH)
            _emit_slots(mi, cnt, splats, xsv, ysv, zsv, nbv, relv, None, 0,
                        r, k)
            return 0

        lax.fori_loop(0, RC, per_ref, 0)
        row_base = (b * M + base_m + sub * RC) * k
        pltpu.sync_copy(relv, xrel_h.at[pl.ds(row_base * 8, RC * k * 8)])


def _sc_scale1(x, y, z, idx1, k, r):
    B, N = x.shape
    M = idx1.shape[1]
    MC = M // (_NW // B)
    RC = min(MC, 64)
    mesh = plsc.VectorSubcoreMesh(core_axis_name="c", subcore_axis_name="s")
    f = pl.kernel(
        functools.partial(_sc_scale1_body, B=B, N=N, M=M, k=k, r=r, MC=MC,
                          RC=RC),
        out_type=[
            jax.ShapeDtypeStruct((B * M * k * 8,), jnp.float32),
            jax.ShapeDtypeStruct((B, M), jnp.float32),
            jax.ShapeDtypeStruct((B, M), jnp.float32),
            jax.ShapeDtypeStruct((B, M), jnp.float32),
        ],
        mesh=mesh,
        compiler_params=pltpu.CompilerParams(needs_layout_passes=False),
        scratch_types=[
            pltpu.VMEM((N,), jnp.float32),
            pltpu.VMEM((N,), jnp.float32),
            pltpu.VMEM((N,), jnp.float32),
            pltpu.VMEM((MC,), jnp.int32),
            pltpu.VMEM((MC,), jnp.float32),
            pltpu.VMEM((MC,), jnp.float32),
            pltpu.VMEM((MC,), jnp.float32),
            pltpu.VMEM((k,), jnp.int32),
            pltpu.VMEM((RC * k * 8,), jnp.float32),
        ],
    )
    xrel, sx, sy, sz = f(x, y, z, idx1)
    return xrel.reshape(B * M * k, 8), sx, sy, sz


def _sc_scalen_body(x_h, y_h, z_h, rx_h, ry_h, rz_h, ff_h,
                    xrel_h, xfeat_h,
                    xsv, ysv, zsv, rxv, ryv, rzv, nbv, relv, idxbv, fbv, sem,
                    *, B, N, M, k, r, C, MC, RC, G):
    cid = lax.axis_index("c")
    sid = lax.axis_index("s")
    wid = sid * 2 + cid
    cpb = _NW // B
    b = wid // cpb
    ch = wid % cpb
    base_m = ch * MC
    pltpu.sync_copy(x_h.at[b], xsv)
    pltpu.sync_copy(y_h.at[b], ysv)
    pltpu.sync_copy(z_h.at[b], zsv)
    pltpu.sync_copy(rx_h.at[b, pl.ds(base_m, MC)], rxv)
    pltpu.sync_copy(ry_h.at[b, pl.ds(base_m, MC)], ryv)
    pltpu.sync_copy(rz_h.at[b, pl.ds(base_m, MC)], rzv)

    NCH = N // 16
    boff = b * N
    for sub in range(MC // RC):
        def per_ref(mi, _):
            m = sub * RC + mi
            cnt, splats = _select_ref(m, rxv, ryv, rzv, xsv, ysv, zsv, nbv,
                                      r, k, NCH)
            _emit_slots(mi, cnt, splats, xsv, ysv, zsv, nbv, relv, idxbv,
                        boff, r, k)
            return 0

        lax.fori_loop(0, RC, per_ref, 0)
        row_base = (b * M + base_m + sub * RC) * k
        pltpu.sync_copy(relv, xrel_h.at[pl.ds(row_base * 8, RC * k * 8)])

        def gath(g, _):
            pltpu.async_copy(ff_h.at[idxbv.at[pl.ds(g * G, G)]], fbv,
                             sem).wait()
            pltpu.sync_copy(fbv, xfeat_h.at[pl.ds(row_base + g * G, G)])
            return 0

        lax.fori_loop(0, (RC * k) // G, gath, 0)


def _sc_scalen(x, y, z, rx, ry, rz, featflat, k, r):
    B, N = x.shape
    M = rx.shape[1]
    C = featflat.shape[1]
    MC = M // (_NW // B)
    RC = min(MC, 128)
    G = 128
    mesh = plsc.VectorSubcoreMesh(core_axis_name="c", subcore_axis_name="s")
    f = pl.kernel(
        functools.partial(_sc_scalen_body, B=B, N=N, M=M, k=k, r=r, C=C,
                          MC=MC, RC=RC, G=G),
        out_type=[
            jax.ShapeDtypeStruct((B * M * k * 8,), jnp.float32),
            jax.ShapeDtypeStruct((B * M * k, C), jnp.float32),
        ],
        mesh=mesh,
        compiler_params=pltpu.CompilerParams(needs_layout_passes=False),
        scratch_types=[
            pltpu.VMEM((N,), jnp.float32),
            pltpu.VMEM((N,), jnp.float32),
            pltpu.VMEM((N,), jnp.float32),
            pltpu.VMEM((MC,), jnp.float32),
            pltpu.VMEM((MC,), jnp.float32),
            pltpu.VMEM((MC,), jnp.float32),
            pltpu.VMEM((k,), jnp.int32),
            pltpu.VMEM((RC * k * 8,), jnp.float32),
            pltpu.VMEM((RC * k,), jnp.int32),
            pltpu.VMEM((G, C), jnp.float32),
            pltpu.SemaphoreType.DMA,
        ],
    )
    xrel, xfeat = f(x, y, z, rx, ry, rz, featflat)
    return xrel.reshape(B * M * k, 8), xfeat


def _mlp1_body(x_ref, w1_ref, b1_ref, w2_ref, b2_ref, w3_ref, b3_ref, o_ref,
               *, RB, k):
    h = jnp.maximum(
        jnp.dot(x_ref[...], w1_ref[...], preferred_element_type=jnp.float32)
        + b1_ref[...], 0.0)
    h = jnp.maximum(
        jnp.dot(h, w2_ref[...], preferred_element_type=jnp.float32)
        + b2_ref[...], 0.0)
    h = jnp.maximum(
        jnp.dot(h, w3_ref[...], preferred_element_type=jnp.float32)
        + b3_ref[...], 0.0)
    C = h.shape[1]
    o_ref[...] = jnp.max(h.reshape(RB, k, C), axis=1)


def _mlp_scale1(xrel, k, layers):
    (w1, b1), (w2, b2), (w3, b3) = layers
    w1p = jnp.concatenate([w1, jnp.zeros((5, w1.shape[1]), jnp.float32)], 0)
    rows = xrel.shape[0]
    refs = rows // k
    RB = 64
    C = w3.shape[1]
    grid = (refs // RB,)
    return pl.pallas_call(
        functools.partial(_mlp1_body, RB=RB, k=k),
        interpret=_INTERP,
        grid=grid,
        in_specs=[
            pl.BlockSpec((RB * k, 8), lambda i: (i, 0)),
            pl.BlockSpec(w1p.shape, lambda i: (0, 0)),
            pl.BlockSpec((1, b1.shape[0]), lambda i: (0, 0)),
            pl.BlockSpec(w2.shape, lambda i: (0, 0)),
            pl.BlockSpec((1, b2.shape[0]), lambda i: (0, 0)),
            pl.BlockSpec(w3.shape, lambda i: (0, 0)),
            pl.BlockSpec((1, b3.shape[0]), lambda i: (0, 0)),
        ],
        out_specs=pl.BlockSpec((RB, C), lambda i: (i, 0)),
        out_shape=jax.ShapeDtypeStruct((refs, C), jnp.float32),
    )(xrel, w1p, b1[None, :], w2, b2[None, :], w3, b3[None, :])


def _mlpn_body(xr_ref, xf_ref, wa_ref, wb_ref, b1_ref, w2_ref, b2_ref,
               w3_ref, b3_ref, o_ref, *, RB, k):
    h = jnp.dot(xr_ref[...], wa_ref[...], preferred_element_type=jnp.float32)
    h = h + jnp.dot(xf_ref[...], wb_ref[...],
                    preferred_element_type=jnp.float32)
    h = jnp.maximum(h + b1_ref[...], 0.0)
    h = jnp.maximum(
        jnp.dot(h, w2_ref[...], preferred_element_type=jnp.float32)
        + b2_ref[...], 0.0)
    h = jnp.maximum(
        jnp.dot(h, w3_ref[...], preferred_element_type=jnp.float32)
        + b3_ref[...], 0.0)
    C = h.shape[1]
    o_ref[...] = jnp.max(h.reshape(RB, k, C), axis=1)


def _mlp_scalen(xrel, xfeat, k, layers):
    (w1, b1), (w2, b2), (w3, b3) = layers
    wa = jnp.concatenate([w1[:3], jnp.zeros((5, w1.shape[1]), jnp.float32)], 0)
    wb = w1[3:]
    rows = xrel.shape[0]
    refs = rows // k
    RB = 32
    Cf = xfeat.shape[1]
    C = w3.shape[1]
    grid = (refs // RB,)
    return pl.pallas_call(
        functools.partial(_mlpn_body, RB=RB, k=k),
        interpret=_INTERP,
        grid=grid,
        in_specs=[
            pl.BlockSpec((RB * k, 8), lambda i: (i, 0)),
            pl.BlockSpec((RB * k, Cf), lambda i: (i, 0)),
            pl.BlockSpec(wa.shape, lambda i: (0, 0)),
            pl.BlockSpec(wb.shape, lambda i: (0, 0)),
            pl.BlockSpec((1, b1.shape[0]), lambda i: (0, 0)),
            pl.BlockSpec(w2.shape, lambda i: (0, 0)),
            pl.BlockSpec((1, b2.shape[0]), lambda i: (0, 0)),
            pl.BlockSpec(w3.shape, lambda i: (0, 0)),
            pl.BlockSpec((1, b3.shape[0]), lambda i: (0, 0)),
        ],
        out_specs=pl.BlockSpec((RB, C), lambda i: (i, 0)),
        out_shape=jax.ShapeDtypeStruct((refs, C), jnp.float32),
    )(xrel, xfeat, wa, wb, b1[None, :], w2, b2[None, :], w3, b3[None, :])


def _interp_body(uc_ref, kr_ref, kf_ref, fin_ref, u1a_ref, u1b_ref, b1_ref,
                 u2_ref, b2_ref, u3_ref, b3_ref, o_ref):
    ux = uc_ref[0, :, 0:1]
    uy = uc_ref[0, :, 1:2]
    uz = uc_ref[0, :, 2:3]
    kx = kr_ref[0, 0:1, :]
    ky = kr_ref[0, 1:2, :]
    kz = kr_ref[0, 2:3, :]
    dx = ux - kx
    dy = uy - ky
    dz = uz - kz
    d2 = (dx * dx + dy * dy) + dz * dz
    Mu, Mk = d2.shape
    lane = lax.broadcasted_iota(jnp.int32, (Mu, Mk), 1)
    wsum = jnp.zeros((Mu, 1), jnp.float32)
    ws = []
    idxs = []
    for _ in range(3):
        mj = jnp.min(d2, axis=1, keepdims=True)
        ij = jnp.min(jnp.where(d2 == mj, lane, Mk), axis=1, keepdims=True)
        wj = 1.0 / jnp.maximum(mj, 1e-10)
        ws.append(wj)
        idxs.append(ij)
        wsum = wsum + wj
        d2 = jnp.where(lane == ij, jnp.float32(1e30), d2)
    wmat = jnp.zeros((Mu, Mk), jnp.float32)
    for wj, ij in zip(ws, idxs):
        wmat = wmat + jnp.where(lane == ij, wj / wsum, 0.0)
    itp = jnp.dot(wmat, kf_ref[0], preferred_element_type=jnp.float32)
    h = jnp.dot(fin_ref[0], u1a_ref[...], preferred_element_type=jnp.float32)
    h = h + jnp.dot(itp, u1b_ref[...], preferred_element_type=jnp.float32)
    h = jnp.maximum(h + b1_ref[...], 0.0)
    h = jnp.maximum(
        jnp.dot(h, u2_ref[...], preferred_element_type=jnp.float32)
        + b2_ref[...], 0.0)
    h = jnp.maximum(
        jnp.dot(h, u3_ref[...], preferred_element_type=jnp.float32)
        + b3_ref[...], 0.0)
    o_ref[0] = h


def _interp_mlp(u_cols, k_rows, k_feats, f_in, layers):
    (w1, b1), (w2, b2), (w3, b3) = layers
    Cin = f_in.shape[2]
    u1a = w1[:Cin]
    u1b = w1[Cin:]
    B, Mu, _ = u_cols.shape
    Mk = k_rows.shape[2]
    C = w3.shape[1]
    return pl.pallas_call(
        _interp_body,
        interpret=_INTERP,
        grid=(B,),
        in_specs=[
            pl.BlockSpec((1, Mu, 8), lambda i: (i, 0, 0)),
            pl.BlockSpec((1, 8, Mk), lambda i: (i, 0, 0)),
            pl.BlockSpec((1, Mk, k_feats.shape[2]), lambda i: (i, 0, 0)),
            pl.BlockSpec((1, Mu, Cin), lambda i: (i, 0, 0)),
            pl.BlockSpec(u1a.shape, lambda i: (0, 0)),
            pl.BlockSpec(u1b.shape, lambda i: (0, 0)),
            pl.BlockSpec((1, b1.shape[0]), lambda i: (0, 0)),
            pl.BlockSpec(w2.shape, lambda i: (0, 0)),
            pl.BlockSpec((1, b2.shape[0]), lambda i: (0, 0)),
            pl.BlockSpec(w3.shape, lambda i: (0, 0)),
            pl.BlockSpec((1, b3.shape[0]), lambda i: (0, 0)),
        ],
        out_specs=pl.BlockSpec((1, Mu, C), lambda i: (i, 0, 0)),
        out_shape=jax.ShapeDtypeStruct((B, Mu, C), jnp.float32),
    )(u_cols, k_rows, k_feats, f_in, u1a, u1b, b1[None, :], w2, b2[None, :],
      w3, b3[None, :])


def _fold(layers):
    s = np.sqrt(1.0 + _BN_EPS).astype(np.float32)
    return [(W * (g / jnp.float32(s)), b) for (W, g, b) in layers]


def _cols(x, y, z, M):
    return jnp.concatenate(
        [x[:, :M, None], y[:, :M, None], z[:, :M, None],
         jnp.zeros((x.shape[0], M, 5), jnp.float32)], axis=2)


def _rows3(x, y, z, M):
    B = x.shape[0]
    return jnp.concatenate(
        [x[:, None, :M], y[:, None, :M], z[:, None, :M],
         jnp.zeros((B, 5, M), jnp.float32)], axis=1)


def kernel(points, params):
    B, N, _ = points.shape
    x = points[:, :, 0]
    y = points[:, :, 1]
    z = points[:, :, 2]

    idx1 = _fps(x, y, z, 2048)
    xrel1, sx, sy, sz = _sc_scale1(x, y, z, idx1, k=64, r=0.2)
    ds1 = _fold(params['ds1'])
    f1 = _mlp_scale1(xrel1, 64, ds1)

    ds2 = _fold(params['ds2'])
    xrel2, xfeat2 = _sc_scalen(sx, sy, sz, sx[:, :1024], sy[:, :1024],
                               sz[:, :1024], f1, k=32, r=0.4)
    f2 = _mlp_scalen(xrel2, xfeat2, 32, ds2)

    ds3 = _fold(params['ds3'])
    xrel3, xfeat3 = _sc_scalen(sx[:, :1024], sy[:, :1024], sz[:, :1024],
                               sx[:, :512], sy[:, :512], sz[:, :512],
                               f2, k=16, r=0.8)
    f3 = _mlp_scalen(xrel3, xfeat3, 16, ds3)

    ds4 = _fold(params['ds4'])
    xrel4, xfeat4 = _sc_scalen(sx[:, :512], sy[:, :512], sz[:, :512],
                               sx[:, :256], sy[:, :256], sz[:, :256],
                               f3, k=16, r=1.2)
    f4 = _mlp_scalen(xrel4, xfeat4, 16, ds4)

    us1 = _fold(params['us1'])
    f3u = _interp_mlp(_cols(sx, sy, sz, 512), _rows3(sx, sy, sz, 256),
                      f4.reshape(B, 256, 256), f3.reshape(B, 512, 256), us1)

    us2 = _fold(params['us2'])
    f2u = _interp_mlp(_cols(sx, sy, sz, 1024), _rows3(sx, sy, sz, 512),
                      f3u, f2.reshape(B, 1024, 256), us2)

    scale2_idx = idx1[:, :1024]
    scale2_pts = jnp.stack([sx[:, :1024], sy[:, :1024], sz[:, :1024]],
                           axis=-1)
    return (scale2_idx, scale2_pts, f2u)


# speedup vs baseline: 1.0025x; 1.0025x over previous
import functools

import jax
import jax.numpy as jnp
import numpy as np
from jax import lax
from jax.experimental import pallas as pl
from jax.experimental.pallas import tpu as pltpu
from jax.experimental.pallas import tpu_sc as plsc

_BN_EPS = 1e-05
_NW = 32


def _fps_body(x_ref, y_ref, z_ref, idx_ref):
    B, S, L = x_ref.shape
    N = S * L
    M = idx_ref.shape[1]
    sub_i = lax.broadcasted_iota(jnp.int32, (B, S, L), 1)
    lane_i = lax.broadcasted_iota(jnp.int32, (B, S, L), 2)
    flat_i = sub_i * L + lane_i
    lane_m = lax.broadcasted_iota(jnp.int32, (B, M), 1)
    x = x_ref[...]
    y = y_ref[...]
    z = z_ref[...]
    idx_ref[...] = jnp.zeros((B, M), jnp.int32)

    def red(op, a):
        return op(op(a, axis=2, keepdims=True), axis=1, keepdims=True)

    def body(i, carry):
        last, dists = carry
        sel = flat_i == last
        lx = red(jnp.sum, jnp.where(sel, x, 0.0))
        ly = red(jnp.sum, jnp.where(sel, y, 0.0))
        lz = red(jnp.sum, jnp.where(sel, z, 0.0))
        dx = x - lx
        dy = y - ly
        dz = z - lz
        d = (dx * dx + dy * dy) + dz * dz
        dmin = jnp.minimum(dists, d)
        mx = red(jnp.max, dmin)
        nxt = red(jnp.min, jnp.where(dmin == mx, flat_i, N))
        idx_ref[...] += jnp.where(lane_m == i, nxt[:, 0, :], 0)
        return nxt, dmin

    lax.fori_loop(1, M, body,
                  (jnp.zeros((B, 1, 1), jnp.int32),
                   jnp.full((B, S, L), 1e10, jnp.float32)))


def _fps(x, y, z, M):
    B, N = x.shape
    x3 = x.reshape(B, N // 128, 128)
    y3 = y.reshape(B, N // 128, 128)
    z3 = z.reshape(B, N // 128, 128)
    return pl.pallas_call(
        _fps_body,
        out_shape=jax.ShapeDtypeStruct((B, M), jnp.int32),
    )(x3, y3, z3)


def _iota16():
    return lax.broadcasted_iota(jnp.int32, (16,), 0)


def _splat(v):
    return jnp.full((16,), v, jnp.int32)


def _select_ref(m, rxv, ryv, rzv, xsv, ysv, zsv, nbv, r, k, NCH):
    i16 = _iota16()
    rxs = plsc.load_gather(rxv, [_splat(m)])
    rys = plsc.load_gather(ryv, [_splat(m)])
    rzs = plsc.load_gather(rzv, [_splat(m)])
    nbv[pl.ds(0, 16)] = jnp.zeros((16,), jnp.int32)
    r2 = jnp.float32(r * r)
    U = 8
    NG = NCH // U

    def cond(st):
        cc, cnt = st
        return jnp.logical_and(cnt < k, cc < NG)

    def body(st):
        cc, cnt = st
        base = cc * (16 * U)
        cntv = jnp.full((16,), cnt, jnp.int32)
        total = jnp.zeros((16,), jnp.int32)
        msks = []
        cums = []
        pcs = []
        for u in range(U):
            off = base + u * 16
            xv = xsv[pl.ds(off, 16)]
            yv = ysv[pl.ds(off, 16)]
            zv = zsv[pl.ds(off, 16)]
            dx = rxs - xv
            dy = rys - yv
            dz = rzs - zv
            d2 = (dx * dx + dy * dy) + dz * dz
            msk = jnp.logical_not(d2 > r2)
            mi32 = msk.astype(jnp.int32)
            msks.append(msk)
            cums.append(plsc.cumsum(mi32))
            pcs.append(plsc.all_reduce_population_count(msk))
            total = total + mi32
        run = cntv
        for u in range(U):
            pos = (cums[u] - 1) + run
            okm = jnp.logical_and(msks[u], pos < k)
            plsc.store_scatter(nbv, [pos], base + u * 16 + i16, mask=okm)
            run = run + pcs[u]
        return cc + 1, cnt + jnp.sum(total)

    _, cnt = lax.while_loop(cond, body, (jnp.int32(0), jnp.int32(0)))
    return jnp.minimum(cnt, k), (rxs, rys, rzs)


def _emit_slots(mi, cnt, splats, xsv, ysv, zsv, nbv, relv, idxbv, boff, r, k):
    i16 = _iota16()
    rxs, rys, rzs = splats
    nb0 = plsc.load_gather(nbv, [jnp.zeros((16,), jnp.int32)])
    rinv = jnp.float32(r)
    z16f = jnp.zeros((16,), jnp.float32)
    for j in range(k // 16):
        slot = j * 16 + i16
        raw = nbv[pl.ds(j * 16, 16)]
        idx16 = jnp.where(slot < cnt, raw, nb0)
        xg = plsc.load_gather(xsv, [idx16])
        yg = plsc.load_gather(ysv, [idx16])
        zg = plsc.load_gather(zsv, [idx16])
        xr = (xg - rxs) / rinv
        yr = (yg - rys) / rinv
        zr = (zg - rzs) / rinv
        rows8 = (mi * k + slot) * 8
        plsc.store_scatter(relv, [rows8 + 0], xr)
        plsc.store_scatter(relv, [rows8 + 1], yr)
        plsc.store_scatter(relv, [rows8 + 2], zr)
        for ccol in range(3, 8):
            plsc.store_scatter(relv, [rows8 + ccol], z16f)
        if idxbv is not None:
            idxbv[pl.ds(mi * k + j * 16, 16)] = idx16 + boff


def _sc_scale1_body(x_h, y_h, z_h, idx_h, xrel_h, sx_h, sy_h, sz_h,
                    xsv, ysv, zsv, idxv, rxv, ryv, rzv, nbv, relv,
                    *, B, N, M, k, r, MC, RC):
    cid = lax.axis_index("c")
    sid = lax.axis_index("s")
    wid = sid * 2 + cid
    cpb = _NW // B
    b = wid // cpb
    ch = wid % cpb
    base_m = ch * MC
    pltpu.sync_copy(x_h.at[b], xsv)
    pltpu.sync_copy(y_h.at[b], ysv)
    pltpu.sync_copy(z_h.at[b], zsv)
    pltpu.sync_copy(idx_h.at[b, pl.ds(base_m, MC)], idxv)

    def gref(j, _):
        iv = idxv[pl.ds(j * 16, 16)]
        rxv[pl.ds(j * 16, 16)] = plsc.load_gather(xsv, [iv])
        ryv[pl.ds(j * 16, 16)] = plsc.load_gather(ysv, [iv])
        rzv[pl.ds(j * 16, 16)] = plsc.load_gather(zsv, [iv])
        return 0

    lax.fori_loop(0, MC // 16, gref, 0)
    pltpu.sync_copy(rxv, sx_h.at[b, pl.ds(base_m, MC)])
    pltpu.sync_copy(ryv, sy_h.at[b, pl.ds(base_m, MC)])
    pltpu.sync_copy(rzv, sz_h.at[b, pl.ds(base_m, MC)])

    NCH = N // 16
    for sub in range(MC // RC):
        def per_ref(mi, _):
            m = sub * RC + mi
            cnt, splats = _select_ref(m, rxv, ryv, rzv, xsv, ysv, zsv, nbv,
                                      r, k, NCH)
            _emit_slots(mi, cnt, splats, xsv, ysv, zsv, nbv, relv, None, 0,
                        r, k)
            return 0

        lax.fori_loop(0, RC, per_ref, 0)
        row_base = (b * M + base_m + sub * RC) * k
        pltpu.sync_copy(relv, xrel_h.at[pl.ds(row_base * 8, RC * k * 8)])


def _sc_scale1(x, y, z, idx1, k, r):
    B, N = x.shape
    M = idx1.shape[1]
    MC = M // (_NW // B)
    RC = min(MC, 64)
    mesh = plsc.VectorSubcoreMesh(core_axis_name="c", subcore_axis_name="s")
    f = pl.kernel(
        functools.partial(_sc_scale1_body, B=B, N=N, M=M, k=k, r=r, MC=MC,
                          RC=RC),
        out_type=[
            jax.ShapeDtypeStruct((B * M * k * 8,), jnp.float32),
            jax.ShapeDtypeStruct((B, M), jnp.float32),
            jax.ShapeDtypeStruct((B, M), jnp.float32),
            jax.ShapeDtypeStruct((B, M), jnp.float32),
        ],
        mesh=mesh,
        compiler_params=pltpu.CompilerParams(needs_layout_passes=False),
        scratch_types=[
            pltpu.VMEM((N,), jnp.float32),
            pltpu.VMEM((N,), jnp.float32),
            pltpu.VMEM((N,), jnp.float32),
            pltpu.VMEM((MC,), jnp.int32),
            pltpu.VMEM((MC,), jnp.float32),
            pltpu.VMEM((MC,), jnp.float32),
            pltpu.VMEM((MC,), jnp.float32),
            pltpu.VMEM((k,), jnp.int32),
            pltpu.VMEM((RC * k * 8,), jnp.float32),
        ],
    )
    xrel, sx, sy, sz = f(x, y, z, idx1)
    return xrel.reshape(B * M * k, 8), sx, sy, sz


def _sc_scalen_body(x_h, y_h, z_h, rx_h, ry_h, rz_h, ff_h,
                    xrel_h, xfeat_h,
                    xsv, ysv, zsv, rxv, ryv, rzv, nbv, relv, idxbv, fbv, sem,
                    *, B, N, M, k, r, C, MC, RC, G):
    cid = lax.axis_index("c")
    sid = lax.axis_index("s")
    wid = sid * 2 + cid
    cpb = _NW // B
    b = wid // cpb
    ch = wid % cpb
    base_m = ch * MC
    pltpu.sync_copy(x_h.at[b], xsv)
    pltpu.sync_copy(y_h.at[b], ysv)
    pltpu.sync_copy(z_h.at[b], zsv)
    pltpu.sync_copy(rx_h.at[b, pl.ds(base_m, MC)], rxv)
    pltpu.sync_copy(ry_h.at[b, pl.ds(base_m, MC)], ryv)
    pltpu.sync_copy(rz_h.at[b, pl.ds(base_m, MC)], rzv)

    NCH = N // 16
    boff = b * N
    for sub in range(MC // RC):
        def per_ref(mi, _):
            m = sub * RC + mi
            cnt, splats = _select_ref(m, rxv, ryv, rzv, xsv, ysv, zsv, nbv,
                                      r, k, NCH)
            _emit_slots(mi, cnt, splats, xsv, ysv, zsv, nbv, relv, idxbv,
                        boff, r, k)
            return 0

        lax.fori_loop(0, RC, per_ref, 0)
        row_base = (b * M + base_m + sub * RC) * k
        pltpu.sync_copy(relv, xrel_h.at[pl.ds(row_base * 8, RC * k * 8)])

        def gath(g, _):
            pltpu.async_copy(ff_h.at[idxbv.at[pl.ds(g * G, G)]], fbv,
                             sem).wait()
            pltpu.sync_copy(fbv, xfeat_h.at[pl.ds(row_base + g * G, G)])
            return 0

        lax.fori_loop(0, (RC * k) // G, gath, 0)


def _sc_scalen(x, y, z, rx, ry, rz, featflat, k, r):
    B, N = x.shape
    M = rx.shape[1]
    C = featflat.shape[1]
    MC = M // (_NW // B)
    RC = min(MC, 128)
    G = 128
    mesh = plsc.VectorSubcoreMesh(core_axis_name="c", subcore_axis_name="s")
    f = pl.kernel(
        functools.partial(_sc_scalen_body, B=B, N=N, M=M, k=k, r=r, C=C,
                          MC=MC, RC=RC, G=G),
        out_type=[
            jax.ShapeDtypeStruct((B * M * k * 8,), jnp.float32),
            jax.ShapeDtypeStruct((B * M * k, C), jnp.float32),
        ],
        mesh=mesh,
        compiler_params=pltpu.CompilerParams(needs_layout_passes=False),
        scratch_types=[
            pltpu.VMEM((N,), jnp.float32),
            pltpu.VMEM((N,), jnp.float32),
            pltpu.VMEM((N,), jnp.float32),
            pltpu.VMEM((MC,), jnp.float32),
            pltpu.VMEM((MC,), jnp.float32),
            pltpu.VMEM((MC,), jnp.float32),
            pltpu.VMEM((k,), jnp.int32),
            pltpu.VMEM((RC * k * 8,), jnp.float32),
            pltpu.VMEM((RC * k,), jnp.int32),
            pltpu.VMEM((G, C), jnp.float32),
            pltpu.SemaphoreType.DMA,
        ],
    )
    xrel, xfeat = f(x, y, z, rx, ry, rz, featflat)
    return xrel.reshape(B * M * k, 8), xfeat


def _mlp1_body(x_ref, w1_ref, b1_ref, w2_ref, b2_ref, w3_ref, b3_ref, o_ref,
               *, RB, k):
    h = jnp.maximum(
        jnp.dot(x_ref[...], w1_ref[...], preferred_element_type=jnp.float32)
        + b1_ref[...], 0.0)
    h = jnp.maximum(
        jnp.dot(h, w2_ref[...], preferred_element_type=jnp.float32)
        + b2_ref[...], 0.0)
    h = jnp.maximum(
        jnp.dot(h, w3_ref[...], preferred_element_type=jnp.float32)
        + b3_ref[...], 0.0)
    C = h.shape[1]
    o_ref[...] = jnp.max(h.reshape(RB, k, C), axis=1)


def _mlp_scale1(xrel, k, layers):
    (w1, b1), (w2, b2), (w3, b3) = layers
    w1p = jnp.concatenate([w1, jnp.zeros((5, w1.shape[1]), jnp.float32)], 0)
    rows = xrel.shape[0]
    refs = rows // k
    RB = 64
    C = w3.shape[1]
    grid = (refs // RB,)
    return pl.pallas_call(
        functools.partial(_mlp1_body, RB=RB, k=k),
        grid=grid,
        in_specs=[
            pl.BlockSpec((RB * k, 8), lambda i: (i, 0)),
            pl.BlockSpec(w1p.shape, lambda i: (0, 0)),
            pl.BlockSpec((1, b1.shape[0]), lambda i: (0, 0)),
            pl.BlockSpec(w2.shape, lambda i: (0, 0)),
            pl.BlockSpec((1, b2.shape[0]), lambda i: (0, 0)),
            pl.BlockSpec(w3.shape, lambda i: (0, 0)),
            pl.BlockSpec((1, b3.shape[0]), lambda i: (0, 0)),
        ],
        out_specs=pl.BlockSpec((RB, C), lambda i: (i, 0)),
        out_shape=jax.ShapeDtypeStruct((refs, C), jnp.float32),
    )(xrel, w1p, b1[None, :], w2, b2[None, :], w3, b3[None, :])


def _mlpn_body(xr_ref, xf_ref, wa_ref, wb_ref, b1_ref, w2_ref, b2_ref,
               w3_ref, b3_ref, o_ref, *, RB, k):
    h = jnp.dot(xr_ref[...], wa_ref[...], preferred_element_type=jnp.float32)
    h = h + jnp.dot(xf_ref[...], wb_ref[...],
                    preferred_element_type=jnp.float32)
    h = jnp.maximum(h + b1_ref[...], 0.0)
    h = jnp.maximum(
        jnp.dot(h, w2_ref[...], preferred_element_type=jnp.float32)
        + b2_ref[...], 0.0)
    h = jnp.maximum(
        jnp.dot(h, w3_ref[...], preferred_element_type=jnp.float32)
        + b3_ref[...], 0.0)
    C = h.shape[1]
    o_ref[...] = jnp.max(h.reshape(RB, k, C), axis=1)


def _mlp_scalen(xrel, xfeat, k, layers):
    (w1, b1), (w2, b2), (w3, b3) = layers
    wa = jnp.concatenate([w1[:3], jnp.zeros((5, w1.shape[1]), jnp.float32)], 0)
    wb = w1[3:]
    rows = xrel.shape[0]
    refs = rows // k
    RB = 32
    Cf = xfeat.shape[1]
    C = w3.shape[1]
    grid = (refs // RB,)
    return pl.pallas_call(
        functools.partial(_mlpn_body, RB=RB, k=k),
        grid=grid,
        in_specs=[
            pl.BlockSpec((RB * k, 8), lambda i: (i, 0)),
            pl.BlockSpec((RB * k, Cf), lambda i: (i, 0)),
            pl.BlockSpec(wa.shape, lambda i: (0, 0)),
            pl.BlockSpec(wb.shape, lambda i: (0, 0)),
            pl.BlockSpec((1, b1.shape[0]), lambda i: (0, 0)),
            pl.BlockSpec(w2.shape, lambda i: (0, 0)),
            pl.BlockSpec((1, b2.shape[0]), lambda i: (0, 0)),
            pl.BlockSpec(w3.shape, lambda i: (0, 0)),
            pl.BlockSpec((1, b3.shape[0]), lambda i: (0, 0)),
        ],
        out_specs=pl.BlockSpec((RB, C), lambda i: (i, 0)),
        out_shape=jax.ShapeDtypeStruct((refs, C), jnp.float32),
    )(xrel, xfeat, wa, wb, b1[None, :], w2, b2[None, :], w3, b3[None, :])


def _interp_body(uc_ref, kr_ref, kf_ref, fin_ref, u1a_ref, u1b_ref, b1_ref,
                 u2_ref, b2_ref, u3_ref, b3_ref, o_ref):
    ux = uc_ref[0, :, 0:1]
    uy = uc_ref[0, :, 1:2]
    uz = uc_ref[0, :, 2:3]
    kx = kr_ref[0, 0:1, :]
    ky = kr_ref[0, 1:2, :]
    kz = kr_ref[0, 2:3, :]
    dx = ux - kx
    dy = uy - ky
    dz = uz - kz
    d2 = (dx * dx + dy * dy) + dz * dz
    Mu, Mk = d2.shape
    lane = lax.broadcasted_iota(jnp.int32, (Mu, Mk), 1)
    wsum = jnp.zeros((Mu, 1), jnp.float32)
    ws = []
    idxs = []
    for _ in range(3):
        mj = jnp.min(d2, axis=1, keepdims=True)
        ij = jnp.min(jnp.where(d2 == mj, lane, Mk), axis=1, keepdims=True)
        wj = 1.0 / jnp.maximum(mj, 1e-10)
        ws.append(wj)
        idxs.append(ij)
        wsum = wsum + wj
        d2 = jnp.where(lane == ij, jnp.float32(1e30), d2)
    wmat = jnp.zeros((Mu, Mk), jnp.float32)
    for wj, ij in zip(ws, idxs):
        wmat = wmat + jnp.where(lane == ij, wj / wsum, 0.0)
    itp = jnp.dot(wmat, kf_ref[0], preferred_element_type=jnp.float32)
    h = jnp.dot(fin_ref[0], u1a_ref[...], preferred_element_type=jnp.float32)
    h = h + jnp.dot(itp, u1b_ref[...], preferred_element_type=jnp.float32)
    h = jnp.maximum(h + b1_ref[...], 0.0)
    h = jnp.maximum(
        jnp.dot(h, u2_ref[...], preferred_element_type=jnp.float32)
        + b2_ref[...], 0.0)
    h = jnp.maximum(
        jnp.dot(h, u3_ref[...], preferred_element_type=jnp.float32)
        + b3_ref[...], 0.0)
    o_ref[0] = h


def _interp_mlp(u_cols, k_rows, k_feats, f_in, layers):
    (w1, b1), (w2, b2), (w3, b3) = layers
    Cin = f_in.shape[2]
    u1a = w1[:Cin]
    u1b = w1[Cin:]
    B, Mu, _ = u_cols.shape
    Mk = k_rows.shape[2]
    C = w3.shape[1]
    return pl.pallas_call(
        _interp_body,
        grid=(B,),
        in_specs=[
            pl.BlockSpec((1, Mu, 8), lambda i: (i, 0, 0)),
            pl.BlockSpec((1, 8, Mk), lambda i: (i, 0, 0)),
            pl.BlockSpec((1, Mk, k_feats.shape[2]), lambda i: (i, 0, 0)),
            pl.BlockSpec((1, Mu, Cin), lambda i: (i, 0, 0)),
            pl.BlockSpec(u1a.shape, lambda i: (0, 0)),
            pl.BlockSpec(u1b.shape, lambda i: (0, 0)),
            pl.BlockSpec((1, b1.shape[0]), lambda i: (0, 0)),
            pl.BlockSpec(w2.shape, lambda i: (0, 0)),
            pl.BlockSpec((1, b2.shape[0]), lambda i: (0, 0)),
            pl.BlockSpec(w3.shape, lambda i: (0, 0)),
            pl.BlockSpec((1, b3.shape[0]), lambda i: (0, 0)),
        ],
        out_specs=pl.BlockSpec((1, Mu, C), lambda i: (i, 0, 0)),
        out_shape=jax.ShapeDtypeStruct((B, Mu, C), jnp.float32),
    )(u_cols, k_rows, k_feats, f_in, u1a, u1b, b1[None, :], w2, b2[None, :],
      w3, b3[None, :])


def _fold(layers):
    s = np.sqrt(1.0 + _BN_EPS).astype(np.float32)
    return [(W * (g / jnp.float32(s)), b) for (W, g, b) in layers]


def _cols(x, y, z, M):
    return jnp.concatenate(
        [x[:, :M, None], y[:, :M, None], z[:, :M, None],
         jnp.zeros((x.shape[0], M, 5), jnp.float32)], axis=2)


def _rows3(x, y, z, M):
    B = x.shape[0]
    return jnp.concatenate(
        [x[:, None, :M], y[:, None, :M], z[:, None, :M],
         jnp.zeros((B, 5, M), jnp.float32)], axis=1)


def kernel(points, params):
    B, N, _ = points.shape
    x = points[:, :, 0]
    y = points[:, :, 1]
    z = points[:, :, 2]

    idx1 = _fps(x, y, z, 2048)
    xrel1, sx, sy, sz = _sc_scale1(x, y, z, idx1, k=64, r=0.2)
    ds1 = _fold(params['ds1'])
    f1 = _mlp_scale1(xrel1, 64, ds1)

    ds2 = _fold(params['ds2'])
    xrel2, xfeat2 = _sc_scalen(sx, sy, sz, sx[:, :1024], sy[:, :1024],
                               sz[:, :1024], f1, k=32, r=0.4)
    f2 = _mlp_scalen(xrel2, xfeat2, 32, ds2)

    ds3 = _fold(params['ds3'])
    xrel3, xfeat3 = _sc_scalen(sx[:, :1024], sy[:, :1024], sz[:, :1024],
                               sx[:, :512], sy[:, :512], sz[:, :512],
                               f2, k=16, r=0.8)
    f3 = _mlp_scalen(xrel3, xfeat3, 16, ds3)

    ds4 = _fold(params['ds4'])
    xrel4, xfeat4 = _sc_scalen(sx[:, :512], sy[:, :512], sz[:, :512],
                               sx[:, :256], sy[:, :256], sz[:, :256],
                               f3, k=16, r=1.2)
    f4 = _mlp_scalen(xrel4, xfeat4, 16, ds4)

    us1 = _fold(params['us1'])
    f3u = _interp_mlp(_cols(sx, sy, sz, 512), _rows3(sx, sy, sz, 256),
                      f4.reshape(B, 256, 256), f3.reshape(B, 512, 256), us1)

    us2 = _fold(params['us2'])
    f2u = _interp_mlp(_cols(sx, sy, sz, 1024), _rows3(sx, sy, sz, 512),
                      f3u, f2.reshape(B, 1024, 256), us2)

    scale2_idx = idx1[:, :1024]
    scale2_pts = jnp.stack([sx[:, :1024], sy[:, :1024], sz[:, :1024]],
                           axis=-1)
    return (scale2_idx, scale2_pts, f2u)
